# baseline (speedup 1.0000x reference)
import functools

import jax
import jax.numpy as jnp
from jax import lax
from jax.experimental import pallas as pl
from jax.experimental.pallas import tpu as pltpu
from jax.experimental.pallas import tpu_sc as plsc

D = 192
DD = D * D
N = D * DD
ROWS = N // 128
DUMMY = N
M = N + 128

BR = 512

TILES = 16
PER_TILE = N // TILES
CHUNK_E = 36864
CHUNK_R = CHUNK_E // 128
NCHUNK = PER_TILE // CHUNK_E
FILL_IT = CHUNK_E // 16


def _tc_idx_body(qp_ref, x_ref, idx_ref):
    w = qp_ref[0]
    qx = qp_ref[1]
    qy = qp_ref[2]
    qz = qp_ref[3]

    r0 = pl.program_id(0) * BR
    f = (r0 + lax.broadcasted_iota(jnp.int32, (BR, 128), 0)) * 128 + \
        lax.broadcasted_iota(jnp.int32, (BR, 128), 1)
    ci = lax.shift_right_logical(lax.shift_right_logical(f, 12) * 58255, 19)
    rem = f - ci * DD
    cj = lax.shift_right_logical(lax.shift_right_logical(rem, 6) * 43691, 17)
    ck = rem - cj * D

    cx = ci.astype(jnp.float32)
    cy = cj.astype(jnp.float32)
    cz = ck.astype(jnp.float32)

    cqw = ((0.0 - qx * cx) - qy * cy) - qz * cz
    cqx = (w * cx + qy * cz) - qz * cy
    cqy = (w * cy + qz * cx) - qx * cz
    cqz = (w * cz + qx * cy) - qy * cx
    nqx = -qx
    nqy = -qy
    nqz = -qz
    rqx = (cqw * nqx + cqx * w + cqy * nqz) - cqz * nqy
    rqy = (cqw * nqy + cqy * w + cqz * nqx) - cqx * nqz
    rqz = (cqw * nqz + cqz * w + cqx * nqy) - cqy * nqx

    ri0 = rqx.astype(jnp.int32)
    ri1 = rqy.astype(jnp.int32)
    ri2 = rqz.astype(jnp.int32)
    valid = ((ri0 >= 0) & (ri0 < D) & (ri1 >= 0) & (ri1 < D)
             & (ri2 >= 0) & (ri2 < D) & (x_ref[...] != 0.0))
    t = ri0 * DD + ri1 * D + ri2
    idx_ref[...] = jnp.where(valid, t, DUMMY)


def _tc_idx(qp, x2d):
    return pl.pallas_call(
        _tc_idx_body,
        grid=(ROWS // BR,),
        in_specs=[
            pl.BlockSpec(memory_space=pltpu.SMEM),
            pl.BlockSpec((BR, 128), lambda i: (i, 0)),
        ],
        out_specs=pl.BlockSpec((BR, 128), lambda i: (i, 0)),
        out_shape=jax.ShapeDtypeStruct((ROWS, 128), jnp.int32),
        compiler_params=pltpu.CompilerParams(
            dimension_semantics=("arbitrary",)),
    )(qp, x2d)


_SC_MESH = plsc.VectorSubcoreMesh(
    core_axis_name="c", subcore_axis_name="s", num_cores=2)

HALF = N // 2
ZPT = HALF // TILES


@functools.partial(
    pl.kernel,
    out_type=jax.ShapeDtypeStruct((M,), jnp.float32),
    mesh=_SC_MESH,
    scratch_types=[
        pltpu.VMEM((CHUNK_E,), jnp.int32),
        pltpu.VMEM((CHUNK_E,), jnp.float32),
    ],
)
def _sc_scatter(idx_hbm, out_hbm, idxbuf, fbuf):
    cid = lax.axis_index("c")
    sid = lax.axis_index("s")

    def _fill(val):
        def body(q, carry):
            fbuf[pl.ds(q * 16, 16)] = jnp.full((16,), val, jnp.float32)
            return carry
        lax.fori_loop(0, FILL_IT, body, 0)

    lo = cid * HALF
    dummy = N + (sid * 2 + cid) * 2

    _fill(0.0)
    for z in range(ZPT // CHUNK_E):
        pltpu.sync_copy(
            fbuf, out_hbm.at[pl.ds(lo + sid * ZPT + z * CHUNK_E, CHUNK_E)])
    plsc.subcore_barrier()

    _fill(1.0)
    tilebase = sid * PER_TILE

    def cbody(c, carry):
        pltpu.sync_copy(
            idx_hbm.at[pl.ds(tilebase + c * CHUNK_E, CHUNK_E)], idxbuf)

        def rbody(q, inner):
            for u in range(4):
                sl = pl.ds((q * 4 + u) * 16, 16)
                v = idxbuf[sl]
                ok = (v >= lo) & (v < lo + HALF)
                idxbuf[sl] = jnp.where(ok, v, dummy)
            return inner
        lax.fori_loop(0, FILL_IT // 4, rbody, 0)
        pltpu.sync_copy(fbuf, out_hbm.at[idxbuf])
        return carry
    lax.fori_loop(0, NCHUNK, cbody, 0)


@jax.jit
def kernel(x, axis, theta):
    sin_half = jnp.sin(theta / 2.0)
    cos_half = jnp.cos(theta / 2.0)
    qp = jnp.concatenate(
        (jnp.reshape(cos_half, (1,)), axis * sin_half)).astype(jnp.float32)
    x2d = x.reshape(ROWS, 128)
    idx2d = _tc_idx(qp, x2d)
    out = _sc_scatter(idx2d.reshape(N))
    return out[:N].reshape(D, D, D)


# speedup vs baseline: 2840.9854x; 2840.9854x over previous
import functools

import jax
import jax.numpy as jnp
from jax import lax
from jax.experimental import pallas as pl
from jax.experimental.pallas import tpu as pltpu
from jax.experimental.pallas import tpu_sc as plsc

D = 192
DD = D * D
N = D * DD
ROWS = N // 128

BR = 512

TILES = 16
PER_TILE = N // TILES
CHUNK_E = 9216
NCHUNK = PER_TILE // CHUNK_E
FILL_IT = CHUNK_E // 16

NSEC = 4
QN = N // NSEC
QSH = QN + 128
QPT = QN // TILES


def _tc_idx_body(qp_ref, x_ref, idx_ref):
    w = qp_ref[0]
    qx = qp_ref[1]
    qy = qp_ref[2]
    qz = qp_ref[3]

    r0 = pl.program_id(0) * BR
    lane = lax.broadcasted_iota(jnp.int32, (BR, 128), 1)
    f = (r0 + lax.broadcasted_iota(jnp.int32, (BR, 128), 0)) * 128 + lane
    ci = lax.shift_right_logical(lax.shift_right_logical(f, 12) * 58255, 19)
    rem = f - ci * DD
    cj = lax.shift_right_logical(lax.shift_right_logical(rem, 6) * 43691, 17)
    ck = rem - cj * D

    cx = ci.astype(jnp.float32)
    cy = cj.astype(jnp.float32)
    cz = ck.astype(jnp.float32)

    cqw = ((0.0 - qx * cx) - qy * cy) - qz * cz
    cqx = (w * cx + qy * cz) - qz * cy
    cqy = (w * cy + qz * cx) - qx * cz
    cqz = (w * cz + qx * cy) - qy * cx
    nqx = -qx
    nqy = -qy
    nqz = -qz
    rqx = (cqw * nqx + cqx * w + cqy * nqz) - cqz * nqy
    rqy = (cqw * nqy + cqy * w + cqz * nqx) - cqx * nqz
    rqz = (cqw * nqz + cqz * w + cqx * nqy) - cqy * nqx

    ri0 = rqx.astype(jnp.int32)
    ri1 = rqy.astype(jnp.int32)
    ri2 = rqz.astype(jnp.int32)
    valid = ((ri0 >= 0) & (ri0 < D) & (ri1 >= 0) & (ri1 < D)
             & (ri2 >= 0) & (ri2 < D) & (x_ref[...] != 0.0))
    t = ri0 * DD + ri1 * D + ri2
    dummy = QN + lane
    for s in range(NSEC):
        base = s * QN
        ts = t - base
        live = valid & (ts >= 0) & (ts < QN)
        idx_ref[s] = jnp.where(live, ts, dummy)


def _tc_idx(qp, x2d):
    return pl.pallas_call(
        _tc_idx_body,
        grid=(ROWS // BR,),
        in_specs=[
            pl.BlockSpec(memory_space=pltpu.SMEM),
            pl.BlockSpec((BR, 128), lambda i: (i, 0)),
        ],
        out_specs=pl.BlockSpec((NSEC, BR, 128), lambda i: (0, i, 0)),
        out_shape=jax.ShapeDtypeStruct((NSEC, ROWS, 128), jnp.int32),
        compiler_params=pltpu.CompilerParams(
            dimension_semantics=("arbitrary",)),
    )(qp, x2d)


_SC_MESH = plsc.VectorSubcoreMesh(
    core_axis_name="c", subcore_axis_name="s", num_cores=2)


@functools.partial(
    pl.kernel,
    out_type=jax.ShapeDtypeStruct((N,), jnp.float32),
    mesh=_SC_MESH,
    scratch_types=[
        pltpu.VMEM((CHUNK_E,), jnp.int32),
        pltpu.VMEM((CHUNK_E,), jnp.float32),
        pltpu.VMEM_SHARED((QSH,), jnp.float32),
    ],
)
def _sc_scatter(idx_hbm, out_hbm, idxbuf, fbuf, shared):
    cid = lax.axis_index("c")
    sid = lax.axis_index("s")

    def _fill(val):
        def body(q, carry):
            fbuf[pl.ds(q * 16, 16)] = jnp.full((16,), val, jnp.float32)
            return carry
        lax.fori_loop(0, FILL_IT, body, 0)

    tilebase = sid * PER_TILE

    for p in range(NSEC // 2):
        sec = (NSEC // 2) * cid + p
        outbase = sec * QN
        idxbase = sec * N + tilebase

        _fill(0.0)
        for z in range(QPT // CHUNK_E):
            pltpu.sync_copy(
                fbuf, shared.at[pl.ds(sid * QPT + z * CHUNK_E, CHUNK_E)])
        plsc.subcore_barrier()

        _fill(1.0)

        def cbody(c, carry):
            pltpu.sync_copy(
                idx_hbm.at[pl.ds(idxbase + c * CHUNK_E, CHUNK_E)], idxbuf)
            pltpu.sync_copy(fbuf, shared.at[idxbuf])
            return carry
        lax.fori_loop(0, NCHUNK, cbody, 0)
        plsc.subcore_barrier()

        for z in range(QPT // CHUNK_E):
            off = sid * QPT + z * CHUNK_E
            pltpu.sync_copy(shared.at[pl.ds(off, CHUNK_E)],
                            out_hbm.at[pl.ds(outbase + off, CHUNK_E)])
        plsc.subcore_barrier()


@jax.jit
def kernel(x, axis, theta):
    sin_half = jnp.sin(theta / 2.0)
    cos_half = jnp.cos(theta / 2.0)
    qp = jnp.concatenate(
        (jnp.reshape(cos_half, (1,)), axis * sin_half)).astype(jnp.float32)
    x2d = x.reshape(ROWS, 128)
    idx = _tc_idx(qp, x2d)
    out = _sc_scatter(idx.reshape(NSEC * N))
    return out.reshape(D, D, D)


# speedup vs baseline: 3188.5041x; 1.1223x over previous
import functools

import jax
import jax.numpy as jnp
from jax import lax
from jax.experimental import pallas as pl
from jax.experimental.pallas import tpu as pltpu
from jax.experimental.pallas import tpu_sc as plsc

D = 192
DD = D * D
N = D * DD
ROWS = N // 128

BR = 512

TILES = 16
PER_TILE = N // TILES
CHUNK_E = 6144
NCHUNK = PER_TILE // CHUNK_E
NP = 6
FILL_IT = CHUNK_E // 16

NSEC = 4
QN = N // NSEC
QSH = QN + 128
QPT = QN // TILES


def _tc_idx_body(qp_ref, x_ref, idx_ref):
    w = qp_ref[0]
    qx = qp_ref[1]
    qy = qp_ref[2]
    qz = qp_ref[3]

    r0 = pl.program_id(0) * BR
    lane = lax.broadcasted_iota(jnp.int32, (BR, 128), 1)
    f = (r0 + lax.broadcasted_iota(jnp.int32, (BR, 128), 0)) * 128 + lane
    ci = lax.shift_right_logical(lax.shift_right_logical(f, 12) * 58255, 19)
    rem = f - ci * DD
    cj = lax.shift_right_logical(lax.shift_right_logical(rem, 6) * 43691, 17)
    ck = rem - cj * D

    cx = ci.astype(jnp.float32)
    cy = cj.astype(jnp.float32)
    cz = ck.astype(jnp.float32)

    cqw = ((0.0 - qx * cx) - qy * cy) - qz * cz
    cqx = (w * cx + qy * cz) - qz * cy
    cqy = (w * cy + qz * cx) - qx * cz
    cqz = (w * cz + qx * cy) - qy * cx
    nqx = -qx
    nqy = -qy
    nqz = -qz
    rqx = (cqw * nqx + cqx * w + cqy * nqz) - cqz * nqy
    rqy = (cqw * nqy + cqy * w + cqz * nqx) - cqx * nqz
    rqz = (cqw * nqz + cqz * w + cqx * nqy) - cqy * nqx

    ri0 = rqx.astype(jnp.int32)
    ri1 = rqy.astype(jnp.int32)
    ri2 = rqz.astype(jnp.int32)
    valid = ((ri0 >= 0) & (ri0 < D) & (ri1 >= 0) & (ri1 < D)
             & (ri2 >= 0) & (ri2 < D) & (x_ref[...] != 0.0))
    t = ri0 * DD + ri1 * D + ri2
    dummy = QN + lane
    for s in range(NSEC):
        base = s * QN
        ts = t - base
        live = valid & (ts >= 0) & (ts < QN)
        idx_ref[s] = jnp.where(live, ts, dummy)


def _tc_idx(qp, x2d):
    return pl.pallas_call(
        _tc_idx_body,
        grid=(ROWS // BR,),
        in_specs=[
            pl.BlockSpec(memory_space=pltpu.SMEM),
            pl.BlockSpec((BR, 128), lambda i: (i, 0)),
        ],
        out_specs=pl.BlockSpec((NSEC, BR, 128), lambda i: (0, i, 0)),
        out_shape=jax.ShapeDtypeStruct((NSEC, ROWS, 128), jnp.int32),
        compiler_params=pltpu.CompilerParams(
            dimension_semantics=("arbitrary",)),
    )(qp, x2d)


_SC_MESH = plsc.VectorSubcoreMesh(
    core_axis_name="c", subcore_axis_name="s", num_cores=2)


@functools.partial(
    pl.kernel,
    out_type=jax.ShapeDtypeStruct((N,), jnp.float32),
    mesh=_SC_MESH,
    scratch_types=[
        pltpu.VMEM((CHUNK_E,), jnp.int32),
        pltpu.VMEM((CHUNK_E,), jnp.int32),
        pltpu.VMEM((CHUNK_E,), jnp.float32),
        pltpu.VMEM_SHARED((QSH,), jnp.float32),
        pltpu.SemaphoreType.DMA,
        pltpu.SemaphoreType.DMA,
    ],
)
def _sc_scatter(idx_hbm, out_hbm, ib0, ib1, fbuf, shared, sem0, sem1):
    cid = lax.axis_index("c")
    sid = lax.axis_index("s")

    def _fill(val):
        def body(q, carry):
            fbuf[pl.ds(q * 16, 16)] = jnp.full((16,), val, jnp.float32)
            return carry
        lax.fori_loop(0, FILL_IT, body, 0)

    tilebase = sid * PER_TILE

    for p in range(NSEC // 2):
        sec = (NSEC // 2) * cid + p
        outbase = sec * QN
        idxbase = sec * N + tilebase

        _fill(0.0)
        for z in range(QPT // CHUNK_E):
            pltpu.sync_copy(
                fbuf, shared.at[pl.ds(sid * QPT + z * CHUNK_E, CHUNK_E)])
        plsc.subcore_barrier()

        _fill(1.0)
        bufs = (ib0, ib1)
        sems = (sem0, sem1)

        def gbody(g, carry):
            gb = idxbase + g * (NP * CHUNK_E)
            pltpu.sync_copy(idx_hbm.at[pl.ds(gb, CHUNK_E)], bufs[0])
            hs = [None,
                  pltpu.async_copy(
                      idx_hbm.at[pl.ds(gb + CHUNK_E, CHUNK_E)],
                      bufs[1], sems[1])]
            for u in range(NP):
                b = u % 2
                if hs[b] is not None:
                    hs[b].wait()
                    hs[b] = None
                pltpu.sync_copy(fbuf, shared.at[bufs[b]])
                if u + 2 < NP:
                    hs[b] = pltpu.async_copy(
                        idx_hbm.at[pl.ds(gb + (u + 2) * CHUNK_E, CHUNK_E)],
                        bufs[b], sems[b])
            return carry
        lax.fori_loop(0, NCHUNK // NP, gbody, 0)
        plsc.subcore_barrier()

        for z in range(QPT // CHUNK_E):
            off = sid * QPT + z * CHUNK_E
            pltpu.sync_copy(shared.at[pl.ds(off, CHUNK_E)],
                            out_hbm.at[pl.ds(outbase + off, CHUNK_E)])
        plsc.subcore_barrier()


@jax.jit
def kernel(x, axis, theta):
    sin_half = jnp.sin(theta / 2.0)
    cos_half = jnp.cos(theta / 2.0)
    qp = jnp.concatenate(
        (jnp.reshape(cos_half, (1,)), axis * sin_half)).astype(jnp.float32)
    x2d = x.reshape(ROWS, 128)
    idx = _tc_idx(qp, x2d)
    out = _sc_scatter(idx.reshape(NSEC * N))
    return out.reshape(D, D, D)
